# baseline (device time: 9732 ns/iter reference)
import jax
import jax.numpy as jnp
from jax import lax
from jax.experimental import pallas as pl
from jax.experimental.pallas import tpu as pltpu

N_GLOBAL = 1024
EPS = 1e-5
NC = 4


def kernel(x, gamma, beta):
    m, n = x.shape
    chunk = m // NC

    gamma2 = gamma.reshape(1, n)
    beta2 = beta.reshape(1, n)

    def body(
        x_ref,
        g_ref,
        b_ref,
        o_ref,
        xv_ref,
        ov_ref,
        stats_ref,
        peer_ref,
        in_sems,
        out_sems,
        send_sems,
        recv_sems,
    ):
        my_x = lax.axis_index("x")
        my_y = lax.axis_index("y")
        peer = (my_x, 1 - my_y)

        loads = []
        for c in range(NC):
            dma = pltpu.make_async_copy(
                x_ref.at[pl.ds(c * chunk, chunk), :], xv_ref.at[c], in_sems.at[c]
            )
            dma.start()
            loads.append(dma)

        barrier = pltpu.get_barrier_semaphore()
        pl.semaphore_signal(
            barrier, inc=1, device_id=peer, device_id_type=pl.DeviceIdType.MESH
        )
        pl.semaphore_wait(barrier, 1)

        g = g_ref[:, :]
        b = b_ref[:, :]

        rdmas = []
        for c in range(NC):
            loads[c].wait()
            xc = xv_ref[c, :, :]
            stats_ref[c, 0, :] = jnp.sum(xc, axis=1)
            stats_ref[c, 1, :] = jnp.sum(xc * xc, axis=1)
            rdma = pltpu.make_async_remote_copy(
                src_ref=stats_ref.at[c],
                dst_ref=peer_ref.at[c],
                send_sem=send_sems.at[c],
                recv_sem=recv_sems.at[c],
                device_id=peer,
                device_id_type=pl.DeviceIdType.MESH,
            )
            rdma.start()
            rdmas.append(rdma)

        stores = []
        for c in range(NC):
            rdmas[c].wait_recv()
            tot_s = stats_ref[c, 0, :] + peer_ref[c, 0, :]
            tot_ss = stats_ref[c, 1, :] + peer_ref[c, 1, :]
            mean = tot_s * (1.0 / N_GLOBAL)
            var = tot_ss * (1.0 / N_GLOBAL) - mean * mean
            inv = lax.rsqrt(var + EPS)
            xc = xv_ref[c, :, :]
            ov_ref[c, :, :] = g * ((xc - mean[:, None]) * inv[:, None]) + b
            dma = pltpu.make_async_copy(
                ov_ref.at[c], o_ref.at[pl.ds(c * chunk, chunk), :], out_sems.at[c]
            )
            dma.start()
            stores.append(dma)

        for c in range(NC):
            stores[c].wait()
            rdmas[c].wait_send()

    return pl.pallas_call(
        body,
        out_shape=jax.ShapeDtypeStruct((m, n), jnp.float32),
        in_specs=[
            pl.BlockSpec(memory_space=pl.ANY),
            pl.BlockSpec(memory_space=pltpu.VMEM),
            pl.BlockSpec(memory_space=pltpu.VMEM),
        ],
        out_specs=pl.BlockSpec(memory_space=pl.ANY),
        scratch_shapes=[
            pltpu.VMEM((NC, chunk, n), jnp.float32),
            pltpu.VMEM((NC, chunk, n), jnp.float32),
            pltpu.VMEM((NC, 2, chunk), jnp.float32),
            pltpu.VMEM((NC, 2, chunk), jnp.float32),
            pltpu.SemaphoreType.DMA((NC,)),
            pltpu.SemaphoreType.DMA((NC,)),
            pltpu.SemaphoreType.DMA((NC,)),
            pltpu.SemaphoreType.DMA((NC,)),
        ],
        compiler_params=pltpu.CompilerParams(collective_id=0),
    )(x, gamma2, beta2)


# device time: 9124 ns/iter; 1.0666x vs baseline; 1.0666x over previous
import jax
import jax.numpy as jnp
from jax import lax
from jax.experimental import pallas as pl
from jax.experimental.pallas import tpu as pltpu

N_GLOBAL = 1024
EPS = 1e-5
NC = 8


def kernel(x, gamma, beta):
    m, n = x.shape
    chunk = m // NC

    gamma2 = gamma.reshape(1, n)
    beta2 = beta.reshape(1, n)

    def body(x_ref, g_ref, b_ref, o_ref, stats_ref, peer_ref, send_sems, recv_sems):
        my_x = lax.axis_index("x")
        my_y = lax.axis_index("y")
        peer = (my_x, 1 - my_y)

        barrier = pltpu.get_barrier_semaphore()
        pl.semaphore_signal(
            barrier, inc=1, device_id=peer, device_id_type=pl.DeviceIdType.MESH
        )

        rdmas = []
        for c in range(NC):
            xc = x_ref[c * chunk : (c + 1) * chunk, :]
            stats_ref[c, 0, :] = jnp.sum(xc, axis=1)
            stats_ref[c, 1, :] = jnp.sum(xc * xc, axis=1)
            if c == 0:
                pl.semaphore_wait(barrier, 1)
            rdma = pltpu.make_async_remote_copy(
                src_ref=stats_ref.at[c],
                dst_ref=peer_ref.at[c],
                send_sem=send_sems.at[c],
                recv_sem=recv_sems.at[c],
                device_id=peer,
                device_id_type=pl.DeviceIdType.MESH,
            )
            rdma.start()
            rdmas.append(rdma)

        g = g_ref[:, :]
        b = b_ref[:, :]

        for c in range(NC):
            rdmas[c].wait_recv()
            tot_s = stats_ref[c, 0, :] + peer_ref[c, 0, :]
            tot_ss = stats_ref[c, 1, :] + peer_ref[c, 1, :]
            mean = tot_s * (1.0 / N_GLOBAL)
            var = tot_ss * (1.0 / N_GLOBAL) - mean * mean
            inv = lax.rsqrt(var + EPS)
            xc = x_ref[c * chunk : (c + 1) * chunk, :]
            o_ref[c * chunk : (c + 1) * chunk, :] = (
                g * ((xc - mean[:, None]) * inv[:, None]) + b
            )

        for c in range(NC):
            rdmas[c].wait_send()

    return pl.pallas_call(
        body,
        out_shape=jax.ShapeDtypeStruct((m, n), jnp.float32),
        in_specs=[
            pl.BlockSpec(memory_space=pltpu.VMEM),
            pl.BlockSpec(memory_space=pltpu.VMEM),
            pl.BlockSpec(memory_space=pltpu.VMEM),
        ],
        out_specs=pl.BlockSpec(memory_space=pltpu.VMEM),
        scratch_shapes=[
            pltpu.VMEM((NC, 2, chunk), jnp.float32),
            pltpu.VMEM((NC, 2, chunk), jnp.float32),
            pltpu.SemaphoreType.DMA((NC,)),
            pltpu.SemaphoreType.DMA((NC,)),
        ],
        compiler_params=pltpu.CompilerParams(collective_id=0),
    )(x, gamma2, beta2)


# device time: 9122 ns/iter; 1.0669x vs baseline; 1.0002x over previous
import jax
import jax.numpy as jnp
from jax import lax
from jax.experimental import pallas as pl
from jax.experimental.pallas import tpu as pltpu

N_GLOBAL = 1024
EPS = 1e-5
NC = 8


def kernel(x, gamma, beta):
    m, n = x.shape
    chunk = m // NC

    gamma2 = gamma.reshape(1, n)
    beta2 = beta.reshape(1, n)

    def body(x_ref, g_ref, b_ref, o_ref, stats_ref, peer_ref, send_sems, recv_sems):
        my_x = lax.axis_index("x")
        my_y = lax.axis_index("y")
        peer = (my_x, 1 - my_y)

        barrier = pltpu.get_barrier_semaphore()
        pl.semaphore_signal(
            barrier, inc=1, device_id=peer, device_id_type=pl.DeviceIdType.MESH
        )

        rdmas = []
        for c in range(NC):
            xc = x_ref[c * chunk : (c + 1) * chunk, :]
            stats_ref[c, 0, :] = jnp.sum(xc, axis=1)
            stats_ref[c, 1, :] = jnp.sum(xc * xc, axis=1)
            if c == 0:
                pl.semaphore_wait(barrier, 1)
            rdma = pltpu.make_async_remote_copy(
                src_ref=stats_ref.at[c],
                dst_ref=peer_ref.at[c],
                send_sem=send_sems.at[c],
                recv_sem=recv_sems.at[c],
                device_id=peer,
                device_id_type=pl.DeviceIdType.MESH,
            )
            rdma.start()
            rdmas.append(rdma)

        g = g_ref[:, :]
        b = b_ref[:, :]

        for c in range(NC):
            rdmas[c].wait_recv()
            tot_s = stats_ref[c, 0, :] + peer_ref[c, 0, :]
            tot_ss = stats_ref[c, 1, :] + peer_ref[c, 1, :]
            mean = tot_s * (1.0 / N_GLOBAL)
            var = tot_ss * (1.0 / N_GLOBAL) - mean * mean
            inv = lax.rsqrt(var + EPS)
            shift = -mean * inv
            xc = x_ref[c * chunk : (c + 1) * chunk, :]
            norm = xc * inv[:, None] + shift[:, None]
            o_ref[c * chunk : (c + 1) * chunk, :] = norm * g + b

        for c in range(NC):
            rdmas[c].wait_send()

    return pl.pallas_call(
        body,
        out_shape=jax.ShapeDtypeStruct((m, n), jnp.float32),
        in_specs=[
            pl.BlockSpec(memory_space=pltpu.VMEM),
            pl.BlockSpec(memory_space=pltpu.VMEM),
            pl.BlockSpec(memory_space=pltpu.VMEM),
        ],
        out_specs=pl.BlockSpec(memory_space=pltpu.VMEM),
        scratch_shapes=[
            pltpu.VMEM((NC, 2, chunk), jnp.float32),
            pltpu.VMEM((NC, 2, chunk), jnp.float32),
            pltpu.SemaphoreType.DMA((NC,)),
            pltpu.SemaphoreType.DMA((NC,)),
        ],
        compiler_params=pltpu.CompilerParams(collective_id=0),
    )(x, gamma2, beta2)
